# baseline (speedup 1.0000x reference)
import functools

import jax
import jax.numpy as jnp
from jax.experimental import pallas as pl
from jax.experimental.pallas import tpu as pltpu

G = 64
N_ANCH = 20000
NCH = 160
A = NCH * 128
TOPK = 9
NLEV = 3
INF = float("inf")
BIGIDX = 0x7FFFFFFF


def _atss_kernel(lvl_ref, anch_ref, gt_ref, out_ass_ref, out_iou_ref,
                 d_ref, o_ref, pa_ref):
    lane = jax.lax.broadcasted_iota(jnp.int32, (1, 128), 1)
    gi_col = jax.lax.broadcasted_iota(jnp.int32, (G, 128), 0)

    g_x1 = gt_ref[:, 0:1]
    g_y1 = gt_ref[:, 1:2]
    g_x2 = gt_ref[:, 2:3]
    g_y2 = gt_ref[:, 3:4]
    gcx = (g_x1 + g_x2) * 0.5
    gcy = (g_y1 + g_y2) * 0.5
    area_g = (g_x2 - g_x1) * (g_y2 - g_y1)

    n_lv = [lvl_ref[i] for i in range(NLEV)]
    starts = [jnp.int32(0), n_lv[0], n_lv[0] + n_lv[1]]
    total = n_lv[0] + n_lv[1] + n_lv[2]
    c_hi_pre = (total + 127) // 128

    def build_body(ci, carry):
        ab = anch_ref[ci]
        acx = ab[0:1, :]
        acy = ab[1:2, :]
        aw = ab[2:3, :]
        ah = ab[3:4, :]
        a_x1 = acx - aw * 0.5
        a_y1 = acy - ah * 0.5
        a_x2 = acx + aw * 0.5
        a_y2 = acy + ah * 0.5
        iw = jnp.clip(jnp.minimum(a_x2, g_x2) - jnp.maximum(a_x1, g_x1),
                      0.0, None)
        ih = jnp.clip(jnp.minimum(a_y2, g_y2) - jnp.maximum(a_y1, g_y1),
                      0.0, None)
        inter = iw * ih
        union = jnp.clip(aw * ah + area_g - inter, 1e-6, None)
        o_ref[ci] = inter / union

        @pl.when(ci < c_hi_pre)
        def _():
            d_ref[ci] = jnp.sqrt((acx - gcx) ** 2 + (acy - gcy) ** 2)
        return carry

    jax.lax.fori_loop(0, NCH, build_body, 0)

    m_prev = [jnp.full((G, 1), -INF, jnp.float32) for _ in range(NLEV)]
    i_prev = [jnp.full((G, 1), -1, jnp.int32) for _ in range(NLEV)]
    mt = [jnp.full((G, 1), -INF, jnp.float32) for _ in range(NLEV)]
    it = [jnp.full((G, 1), -1, jnp.int32) for _ in range(NLEV)]
    bounds = []
    for lvl in range(NLEV):
        s = starts[lvl]
        e = s + n_lv[lvl]
        bounds.append((s, e, s // 128, (e + 127) // 128))

    for r in range(TOPK):
        for lvl in range(NLEV):
            s, e, c_lo, c_hi = bounds[lvl]
            mp, ip = m_prev[lvl], i_prev[lvl]

            def ext_body(ci, carry, mp=mp, ip=ip, s=s, e=e):
                acc, acci = carry
                blk = d_ref[ci]
                gio = ci * 128 + lane
                in_lvl = (gio >= s) & (gio < e)
                ok = in_lvl & ((blk > mp) | ((blk == mp) & (gio > ip)))
                bm = jnp.where(ok, blk, INF)
                lt = bm < acc
                acc = jnp.where(lt, bm, acc)
                acci = jnp.where(lt, gio, acci)
                return acc, acci

            acc, acci = jax.lax.fori_loop(
                c_lo, c_hi, ext_body,
                (jnp.full((G, 128), INF, jnp.float32),
                 jnp.zeros((G, 128), jnp.int32)))
            m = jnp.min(acc, axis=1, keepdims=True)
            cand = jnp.where(acc == m, acci, BIGIDX)
            i = jnp.min(cand, axis=1, keepdims=True)
            m_prev[lvl] = m
            i_prev[lvl] = i
            fin = m < INF
            mt[lvl] = jnp.where(fin, m, mt[lvl])
            it[lvl] = jnp.where(fin, i, it[lvl])

    k_tot = (jnp.minimum(TOPK, n_lv[0]) + jnp.minimum(TOPK, n_lv[1])
             + jnp.minimum(TOPK, n_lv[2])).astype(jnp.float32)

    def sel_of(ci, lvl, s, e):
        blk = d_ref[ci]
        gio = ci * 128 + lane
        in_lvl = (gio >= s) & (gio < e)
        return in_lvl & ((blk < mt[lvl])
                         | ((blk == mt[lvl]) & (gio <= it[lvl])))

    s_acc = jnp.zeros((G, 128), jnp.float32)
    for lvl in range(NLEV):
        s, e, c_lo, c_hi = bounds[lvl]

        def sum_body(ci, acc, lvl=lvl, s=s, e=e):
            return acc + jnp.where(sel_of(ci, lvl, s, e), o_ref[ci], 0.0)

        s_acc = jax.lax.fori_loop(c_lo, c_hi, sum_body, s_acc)
    mean = jnp.sum(s_acc, axis=1, keepdims=True) / k_tot

    q_acc = jnp.zeros((G, 128), jnp.float32)
    for lvl in range(NLEV):
        s, e, c_lo, c_hi = bounds[lvl]

        def sq_body(ci, acc, lvl=lvl, s=s, e=e):
            dv = o_ref[ci] - mean
            return acc + jnp.where(sel_of(ci, lvl, s, e), dv * dv, 0.0)

        q_acc = jax.lax.fori_loop(c_lo, c_hi, sq_body, q_acc)
    sq = jnp.sum(q_acc, axis=1, keepdims=True)
    std = jnp.sqrt(sq / jnp.maximum(k_tot - 1.0, 1.0))
    thresh = mean + std

    def pa_zero(ci, carry):
        pa_ref[ci] = jnp.zeros((1, 128), jnp.int32)
        return carry

    jax.lax.fori_loop(0, c_hi_pre, pa_zero, 0)

    for lvl in range(NLEV):
        s, e, c_lo, c_hi = bounds[lvl]

        def pos_body(ci, carry, lvl=lvl, s=s, e=e):
            ab = anch_ref[ci]
            acx = ab[0:1, :]
            acy = ab[1:2, :]
            in_gts = ((acx > g_x1) & (acx < g_x2)
                      & (acy > g_y1) & (acy < g_y2))
            pos = sel_of(ci, lvl, s, e) & (o_ref[ci] >= thresh) & in_gts
            pa_ref[ci] = jnp.maximum(
                pa_ref[ci],
                jnp.max(pos.astype(jnp.int32), axis=0, keepdims=True))
            return carry

        jax.lax.fori_loop(c_lo, c_hi, pos_body, 0)

    def fin_body(ci, carry):
        ob = o_ref[ci]
        mx = jnp.max(ob, axis=0, keepdims=True)
        am = jnp.min(jnp.where(ob == mx, gi_col, G), axis=0, keepdims=True)
        pa = jnp.where(ci < c_hi_pre, pa_ref[jnp.minimum(ci, NCH - 1)], 0)
        out_ass_ref[ci] = jnp.where(pa > 0, am + 1, 0).astype(jnp.int32)
        out_iou_ref[ci] = mx
        return carry

    jax.lax.fori_loop(0, NCH, fin_body, 0)


@functools.partial(jax.jit, static_argnames=())
def _run(anch_c, gt_p, lvl):
    return pl.pallas_call(
        _atss_kernel,
        out_shape=[
            jax.ShapeDtypeStruct((NCH, 1, 128), jnp.int32),
            jax.ShapeDtypeStruct((NCH, 1, 128), jnp.float32),
        ],
        in_specs=[
            pl.BlockSpec(memory_space=pltpu.SMEM),
            pl.BlockSpec(memory_space=pltpu.VMEM),
            pl.BlockSpec(memory_space=pltpu.VMEM),
        ],
        out_specs=[
            pl.BlockSpec(memory_space=pltpu.VMEM),
            pl.BlockSpec(memory_space=pltpu.VMEM),
        ],
        scratch_shapes=[
            pltpu.VMEM((NCH, G, 128), jnp.float32),
            pltpu.VMEM((NCH, G, 128), jnp.float32),
            pltpu.VMEM((NCH, 1, 128), jnp.int32),
        ],
    )(lvl, anch_c, gt_p)


def kernel(anchors, num_anchors_per_level, gt_boxes):
    at = jnp.zeros((4, A), jnp.float32).at[:, :N_ANCH].set(anchors.T)
    anch_c = at.reshape(4, NCH, 128).transpose(1, 0, 2)
    lvl = jnp.stack([jnp.asarray(n, jnp.int32)
                     for n in num_anchors_per_level])
    ass3, miou3 = _run(anch_c, gt_boxes, lvl)
    return (ass3.reshape(A)[:N_ANCH], miou3.reshape(A)[:N_ANCH])


# speedup vs baseline: 1.0399x; 1.0399x over previous
import functools

import jax
import jax.numpy as jnp
from jax.experimental import pallas as pl
from jax.experimental.pallas import tpu as pltpu

G = 64
N_ANCH = 20000
NCH = 160
A = NCH * 128
TOPK = 9
NLEV = 3
INF = float("inf")
BIG_SEL = 1e30
BIGIDX_F = float(A)


def _atss_kernel(lvl_ref, anch_ref, gt_ref, out_ass_ref, out_iou_ref,
                 d_ref, o_ref, pa_ref):
    lane = jax.lax.broadcasted_iota(jnp.int32, (1, 128), 1)
    lane_f = lane.astype(jnp.float32)
    gi_col = jax.lax.broadcasted_iota(jnp.int32, (G, 128), 0)

    g_x1 = gt_ref[:, 0:1]
    g_y1 = gt_ref[:, 1:2]
    g_x2 = gt_ref[:, 2:3]
    g_y2 = gt_ref[:, 3:4]
    gcx = (g_x1 + g_x2) * 0.5
    gcy = (g_y1 + g_y2) * 0.5
    area_g = (g_x2 - g_x1) * (g_y2 - g_y1)

    n_lv = [lvl_ref[i] for i in range(NLEV)]
    starts = [jnp.int32(0), n_lv[0], n_lv[0] + n_lv[1]]
    total = n_lv[0] + n_lv[1] + n_lv[2]
    c_hi_pre = (total + 127) // 128

    def build_body(ci, carry):
        ab = anch_ref[ci]
        acx = ab[0:1, :]
        acy = ab[1:2, :]
        aw = ab[2:3, :]
        ah = ab[3:4, :]
        a_x1 = acx - aw * 0.5
        a_y1 = acy - ah * 0.5
        a_x2 = acx + aw * 0.5
        a_y2 = acy + ah * 0.5
        iw = jnp.clip(jnp.minimum(a_x2, g_x2) - jnp.maximum(a_x1, g_x1),
                      0.0, None)
        ih = jnp.clip(jnp.minimum(a_y2, g_y2) - jnp.maximum(a_y1, g_y1),
                      0.0, None)
        inter = iw * ih
        union = jnp.clip(aw * ah + area_g - inter, 1e-6, None)
        o_ref[ci] = inter / union

        @pl.when(ci < c_hi_pre)
        def _():
            d_ref[ci] = jnp.sqrt((acx - gcx) ** 2 + (acy - gcy) ** 2)
        return carry

    jax.lax.fori_loop(0, NCH, build_body, 0)

    m_prev = [jnp.full((G, 1), -INF, jnp.float32) for _ in range(NLEV)]
    i_prev = [jnp.full((G, 1), -1, jnp.int32) for _ in range(NLEV)]
    mt = [jnp.full((G, 1), -INF, jnp.float32) for _ in range(NLEV)]
    it = [jnp.full((G, 1), -1, jnp.int32) for _ in range(NLEV)]
    bounds = []
    for lvl in range(NLEV):
        s = starts[lvl]
        e = s + n_lv[lvl]
        bounds.append((s, e, s // 128, (e + 127) // 128))

    for r in range(TOPK):
        for lvl in range(NLEV):
            s, e, c_lo, c_hi = bounds[lvl]
            mp, ip = m_prev[lvl], i_prev[lvl]

            def ext_body(ci, carry, mp=mp, ip=ip, s=s, e=e):
                acc, acci = carry
                blk = d_ref[ci]
                gio = ci * 128 + lane
                giof = (ci * 128).astype(jnp.float32) + lane_f
                in_lvl = (gio >= s) & (gio < e)
                ok = in_lvl & ((blk > mp) | ((blk == mp) & (gio > ip)))
                bm = jnp.where(ok, blk, INF)
                lt = bm < acc
                acc = jnp.where(lt, bm, acc)
                acci = jnp.where(lt, giof, acci)
                return acc, acci

            acc, acci = jax.lax.fori_loop(
                c_lo, c_hi, ext_body,
                (jnp.full((G, 128), INF, jnp.float32),
                 jnp.full((G, 128), BIGIDX_F, jnp.float32)))
            m = jnp.min(acc, axis=1, keepdims=True)
            cand = jnp.where(acc == m, acci, BIGIDX_F)
            i = jnp.min(cand, axis=1, keepdims=True).astype(jnp.int32)
            m_prev[lvl] = m
            i_prev[lvl] = i
            fin = m < INF
            mt[lvl] = jnp.where(fin, m, mt[lvl])
            it[lvl] = jnp.where(fin, i, it[lvl])

    for lvl in range(NLEV):
        s, e, c_lo, c_hi = bounds[lvl]
        mtl, itl = mt[lvl], it[lvl]

        def mark_body(ci, carry, mtl=mtl, itl=itl, s=s, e=e):
            blk = d_ref[ci]
            gio = ci * 128 + lane
            in_lvl = (gio >= s) & (gio < e)
            sel = in_lvl & ((blk < mtl) | ((blk == mtl) & (gio <= itl)))
            d_ref[ci] = jnp.where(sel, BIG_SEL, blk)
            return carry

        jax.lax.fori_loop(c_lo, c_hi, mark_body, 0)

    k_tot = (jnp.minimum(TOPK, n_lv[0]) + jnp.minimum(TOPK, n_lv[1])
             + jnp.minimum(TOPK, n_lv[2])).astype(jnp.float32)

    def sum_body(ci, acc):
        sel = d_ref[ci] == BIG_SEL
        return acc + jnp.where(sel, o_ref[ci], 0.0)

    s_acc = jax.lax.fori_loop(0, c_hi_pre, sum_body,
                              jnp.zeros((G, 128), jnp.float32))
    mean = jnp.sum(s_acc, axis=1, keepdims=True) / k_tot

    def sq_body(ci, acc):
        sel = d_ref[ci] == BIG_SEL
        dv = o_ref[ci] - mean
        return acc + jnp.where(sel, dv * dv, 0.0)

    q_acc = jax.lax.fori_loop(0, c_hi_pre, sq_body,
                              jnp.zeros((G, 128), jnp.float32))
    sq = jnp.sum(q_acc, axis=1, keepdims=True)
    std = jnp.sqrt(sq / jnp.maximum(k_tot - 1.0, 1.0))
    thresh = mean + std

    def pos_body(ci, carry):
        ab = anch_ref[ci]
        acx = ab[0:1, :]
        acy = ab[1:2, :]
        in_gts = ((acx > g_x1) & (acx < g_x2)
                  & (acy > g_y1) & (acy < g_y2))
        sel = d_ref[ci] == BIG_SEL
        pos = sel & (o_ref[ci] >= thresh) & in_gts
        pa_ref[ci] = jnp.max(pos.astype(jnp.int32), axis=0, keepdims=True)
        return carry

    jax.lax.fori_loop(0, c_hi_pre, pos_body, 0)

    def fin_body(ci, carry):
        ob = o_ref[ci]
        mx = jnp.max(ob, axis=0, keepdims=True)
        am = jnp.min(jnp.where(ob == mx, gi_col, G), axis=0, keepdims=True)
        pa = jnp.where(ci < c_hi_pre, pa_ref[ci], 0)
        out_ass_ref[ci] = jnp.where(pa > 0, am + 1, 0).astype(jnp.int32)
        out_iou_ref[ci] = mx
        return carry

    jax.lax.fori_loop(0, NCH, fin_body, 0)


@functools.partial(jax.jit, static_argnames=())
def _run(anch_c, gt_p, lvl):
    return pl.pallas_call(
        _atss_kernel,
        out_shape=[
            jax.ShapeDtypeStruct((NCH, 1, 128), jnp.int32),
            jax.ShapeDtypeStruct((NCH, 1, 128), jnp.float32),
        ],
        in_specs=[
            pl.BlockSpec(memory_space=pltpu.SMEM),
            pl.BlockSpec(memory_space=pltpu.VMEM),
            pl.BlockSpec(memory_space=pltpu.VMEM),
        ],
        out_specs=[
            pl.BlockSpec(memory_space=pltpu.VMEM),
            pl.BlockSpec(memory_space=pltpu.VMEM),
        ],
        scratch_shapes=[
            pltpu.VMEM((NCH, G, 128), jnp.float32),
            pltpu.VMEM((NCH, G, 128), jnp.float32),
            pltpu.VMEM((NCH, 1, 128), jnp.int32),
        ],
    )(lvl, anch_c, gt_p)


def kernel(anchors, num_anchors_per_level, gt_boxes):
    at = jnp.zeros((4, A), jnp.float32).at[:, :N_ANCH].set(anchors.T)
    anch_c = at.reshape(4, NCH, 128).transpose(1, 0, 2)
    lvl = jnp.stack([jnp.asarray(n, jnp.int32)
                     for n in num_anchors_per_level])
    ass3, miou3 = _run(anch_c, gt_boxes, lvl)
    return (ass3.reshape(A)[:N_ANCH], miou3.reshape(A)[:N_ANCH])


# speedup vs baseline: 1.1206x; 1.0776x over previous
import functools

import jax
import jax.numpy as jnp
from jax.experimental import pallas as pl
from jax.experimental.pallas import tpu as pltpu

G = 64
N_ANCH = 20000
NCH = 160
A = NCH * 128
TOPK = 9
NLEV = 3
BIG_SEL = 1e30
INF = float("inf")


def _atss_kernel(lvl_ref, anch_ref, gt_ref, out_ass_ref, out_iou_ref,
                 d_ref, o_ref, pa_ref):
    lane = jax.lax.broadcasted_iota(jnp.int32, (1, 128), 1)
    gi_col = jax.lax.broadcasted_iota(jnp.int32, (G, 128), 0)

    g_x1 = gt_ref[:, 0:1]
    g_y1 = gt_ref[:, 1:2]
    g_x2 = gt_ref[:, 2:3]
    g_y2 = gt_ref[:, 3:4]
    gcx = (g_x1 + g_x2) * 0.5
    gcy = (g_y1 + g_y2) * 0.5
    area_g = (g_x2 - g_x1) * (g_y2 - g_y1)

    n0 = lvl_ref[0]
    n1 = lvl_ref[1]
    n2 = lvl_ref[2]
    c_hi_pre = (n0 + n1 + n2 + 127) // 128

    def build_body(ci, carry):
        ab = anch_ref[ci]
        acx = ab[0:1, :]
        acy = ab[1:2, :]
        aw = ab[2:3, :]
        ah = ab[3:4, :]
        a_x1 = acx - aw * 0.5
        a_y1 = acy - ah * 0.5
        a_x2 = acx + aw * 0.5
        a_y2 = acy + ah * 0.5
        iw = jnp.clip(jnp.minimum(a_x2, g_x2) - jnp.maximum(a_x1, g_x1),
                      0.0, None)
        ih = jnp.clip(jnp.minimum(a_y2, g_y2) - jnp.maximum(a_y1, g_y1),
                      0.0, None)
        inter = iw * ih
        union = jnp.clip(aw * ah + area_g - inter, 1e-6, None)
        o_ref[ci] = inter / union

        @pl.when(ci < c_hi_pre)
        def _():
            d_ref[ci] = jnp.sqrt((acx - gcx) ** 2 + (acy - gcy) ** 2)
        return carry

    jax.lax.fori_loop(0, NCH, build_body, 0)

    starts = (jnp.int32(0), n0, n0 + n1)
    sizes = (n0, n1, n2)
    for lvl in range(NLEV):
        s = starts[lvl]
        n = sizes[lvl]
        c_lo = s // 128
        c_hi = (s + n + 127) // 128
        prev_idx = None
        prev_ok = None
        for r in range(TOPK):
            pi, pok = prev_idx, prev_ok

            def ext_body(ci, carry, pi=pi, pok=pok):
                acc, acci = carry
                blk = d_ref[ci]
                gio = ci * 128 + lane
                if pi is not None:
                    mk = (gio == pi) & pok
                    blk = jnp.where(mk, BIG_SEL, blk)
                    d_ref[ci] = blk
                in_lvl = (gio >= s) & (gio < s + n)
                bm = jnp.where(in_lvl, blk, INF)
                lt = bm < acc
                acc = jnp.where(lt, bm, acc)
                acci = jnp.where(lt, ci, acci)
                return acc, acci

            acc, acci = jax.lax.fori_loop(
                c_lo, c_hi, ext_body,
                (jnp.full((G, 128), INF, jnp.float32),
                 jnp.zeros((G, 128), jnp.int32)))
            m = jnp.min(acc, axis=1, keepdims=True)
            cand = jnp.where(acc == m, acci * 128 + lane, A)
            prev_idx = jnp.min(cand, axis=1, keepdims=True)
            prev_ok = r < n

        pi, pok = prev_idx, prev_ok

        def mark_body(ci, carry, pi=pi, pok=pok):
            gio = ci * 128 + lane
            mk = (gio == pi) & pok
            d_ref[ci] = jnp.where(mk, BIG_SEL, d_ref[ci])
            return carry

        jax.lax.fori_loop(c_lo, c_hi, mark_body, 0)

    k_tot = (jnp.minimum(TOPK, n0) + jnp.minimum(TOPK, n1)
             + jnp.minimum(TOPK, n2)).astype(jnp.float32)

    def sum_body(ci, acc):
        sel = d_ref[ci] == BIG_SEL
        return acc + jnp.where(sel, o_ref[ci], 0.0)

    s_acc = jax.lax.fori_loop(0, c_hi_pre, sum_body,
                              jnp.zeros((G, 128), jnp.float32))
    mean = jnp.sum(s_acc, axis=1, keepdims=True) / k_tot

    def sq_body(ci, acc):
        sel = d_ref[ci] == BIG_SEL
        dv = o_ref[ci] - mean
        return acc + jnp.where(sel, dv * dv, 0.0)

    q_acc = jax.lax.fori_loop(0, c_hi_pre, sq_body,
                              jnp.zeros((G, 128), jnp.float32))
    sq = jnp.sum(q_acc, axis=1, keepdims=True)
    std = jnp.sqrt(sq / jnp.maximum(k_tot - 1.0, 1.0))
    thresh = mean + std

    def pos_body(ci, carry):
        ab = anch_ref[ci]
        acx = ab[0:1, :]
        acy = ab[1:2, :]
        in_gts = ((acx > g_x1) & (acx < g_x2)
                  & (acy > g_y1) & (acy < g_y2))
        sel = d_ref[ci] == BIG_SEL
        pos = sel & (o_ref[ci] >= thresh) & in_gts
        pa_ref[ci] = jnp.max(pos.astype(jnp.int32), axis=0, keepdims=True)
        return carry

    jax.lax.fori_loop(0, c_hi_pre, pos_body, 0)

    def fin_body(ci, carry):
        ob = o_ref[ci]
        mx = jnp.max(ob, axis=0, keepdims=True)
        am = jnp.min(jnp.where(ob == mx, gi_col, G), axis=0, keepdims=True)
        pa = jnp.where(ci < c_hi_pre, pa_ref[ci], 0)
        out_ass_ref[ci] = jnp.where(pa > 0, am + 1, 0).astype(jnp.int32)
        out_iou_ref[ci] = mx
        return carry

    jax.lax.fori_loop(0, NCH, fin_body, 0)


@functools.partial(jax.jit, static_argnames=())
def _run(anch_c, gt_p, lvl):
    return pl.pallas_call(
        _atss_kernel,
        out_shape=[
            jax.ShapeDtypeStruct((NCH, 1, 128), jnp.int32),
            jax.ShapeDtypeStruct((NCH, 1, 128), jnp.float32),
        ],
        in_specs=[
            pl.BlockSpec(memory_space=pltpu.SMEM),
            pl.BlockSpec(memory_space=pltpu.VMEM),
            pl.BlockSpec(memory_space=pltpu.VMEM),
        ],
        out_specs=[
            pl.BlockSpec(memory_space=pltpu.VMEM),
            pl.BlockSpec(memory_space=pltpu.VMEM),
        ],
        scratch_shapes=[
            pltpu.VMEM((NCH, G, 128), jnp.float32),
            pltpu.VMEM((NCH, G, 128), jnp.float32),
            pltpu.VMEM((NCH, 1, 128), jnp.int32),
        ],
    )(lvl, anch_c, gt_p)


def kernel(anchors, num_anchors_per_level, gt_boxes):
    at = jnp.zeros((4, A), jnp.float32).at[:, :N_ANCH].set(anchors.T)
    anch_c = at.reshape(4, NCH, 128).transpose(1, 0, 2)
    lvl = jnp.stack([jnp.asarray(n, jnp.int32)
                     for n in num_anchors_per_level])
    ass3, miou3 = _run(anch_c, gt_boxes, lvl)
    return (ass3.reshape(A)[:N_ANCH], miou3.reshape(A)[:N_ANCH])
